# baseline (speedup 1.0000x reference)
import functools

import jax
import jax.numpy as jnp
from jax import lax
from jax.experimental import pallas as pl
from jax.experimental.pallas import tpu as pltpu
from jax.experimental.pallas import tpu_sc as plsc

VOCAB = 100000
EMBED = 64
OUT = 5
B = 16384
L = 50
DP = 8

NC = 2
NS = 16
NW = NC * NS
CHUNK = L
ROWS_PW = B // NW
CHUNKS_PW = ROWS_PW
NBUF = 8
GROUPS = CHUNKS_PW // NBUF


def _proj_body(t_ref, w_ref, z_ref):
    zz = jax.lax.dot_general(
        t_ref[...], w_ref[...],
        dimension_numbers=(((1,), (1,)), ((), ())),
        preferred_element_type=jnp.float32)
    z3 = zz.reshape(_ROWS_BLK // 16, 16, DP)
    packed = jnp.concatenate([z3[:, a, :] for a in range(16)], axis=1)
    z_ref[...] = packed.reshape(1, _ROWS_BLK // 16, 128)


_ROWS_BLK = 2000

_project = pl.pallas_call(
    _proj_body,
    grid=(VOCAB // _ROWS_BLK,),
    in_specs=[
        pl.BlockSpec((_ROWS_BLK, EMBED), lambda i: (i, 0)),
        pl.BlockSpec((DP, EMBED), lambda i: (0, 0)),
    ],
    out_specs=pl.BlockSpec((1, _ROWS_BLK // 16, 128), lambda i: (i, 0, 0)),
    out_shape=jax.ShapeDtypeStruct(
        (VOCAB // _ROWS_BLK, _ROWS_BLK // 16, 128), jnp.float32),
)


_mesh = plsc.VectorSubcoreMesh(core_axis_name="c", subcore_axis_name="s")


@functools.partial(
    pl.kernel,
    out_type=jax.ShapeDtypeStruct((B, 16), jnp.float32),
    mesh=_mesh,
    scratch_types=[
        pltpu.VMEM((ROWS_PW, L), jnp.int32),
        pltpu.VMEM((NBUF * CHUNK, DP), jnp.float32),
        pltpu.VMEM((ROWS_PW, 16), jnp.float32),
        pltpu.VMEM((16,), jnp.float32),
        pltpu.SemaphoreType.DMA,
        [pltpu.SemaphoreType.DMA] * NBUF,
    ],
    compiler_params=pltpu.CompilerParams(use_tc_tiling_on_sc=False, needs_layout_passes=False),
)
def _sc_pool(idx_hbm, z_hbm, lb_hbm, out_hbm, idx_v, gbuf, out_v, lb_v,
             sem_i, gsems):
    wid = lax.axis_index("s") * NC + lax.axis_index("c")
    rbase = wid * ROWS_PW

    pltpu.sync_copy(lb_hbm, lb_v)
    pltpu.sync_copy(idx_hbm.at[pl.ds(rbase, ROWS_PW)], idx_v)

    def _src(j):
        return z_hbm.at[idx_v.at[j]]

    def _dst(k):
        return gbuf.at[pl.ds(k * CHUNK, CHUNK)]

    for k in range(NBUF):
        pltpu.async_copy(_src(k), _dst(k), gsems[k])

    lane = lax.iota(jnp.int32, 16)
    half = lax.shift_right_logical(lane, 3)
    col = lane & 7
    fold = (lane + 8) & 15

    def _sum_rows(k, lo, hi):
        vals = [
            plsc.load_gather(gbuf, [k * CHUNK + 2 * i + half, col])
            for i in range(lo, hi)
        ]
        while len(vals) > 1:
            nxt = [vals[i] + vals[i + 1] for i in range(0, len(vals) - 1, 2)]
            if len(vals) % 2:
                nxt.append(vals[-1])
            vals = nxt
        s = vals[0]
        return s + jnp.take(s, fold, axis=0)

    def _group(g, carry):
        for k in range(NBUF):
            j = g * NBUF + k
            pltpu.make_async_copy(_src(k), _dst(k), gsems[k]).wait()
            out_v[j, :] = _sum_rows(k, 0, L // 2) + lb_v[...]

            @pl.when(j + NBUF < CHUNKS_PW)
            def _():
                pltpu.async_copy(_src(j + NBUF), _dst(k), gsems[k])
        return carry

    lax.fori_loop(0, GROUPS, _group, 0)

    pltpu.sync_copy(out_v, out_hbm.at[pl.ds(rbase, ROWS_PW)])


def kernel(inputs, table, W, b):
    W_pad = jnp.zeros((DP, EMBED), jnp.float32).at[:OUT].set(W)
    lb = jnp.zeros((16,), jnp.float32).at[:OUT].set(jnp.float32(L) * b)
    z = _project(table, W_pad).reshape(VOCAB, DP)
    pooled = _sc_pool(inputs, z, lb)
    return pooled[:, :OUT]


# speedup vs baseline: 1.1678x; 1.1678x over previous
import functools

import jax
import jax.numpy as jnp
from jax import lax
from jax.experimental import pallas as pl
from jax.experimental.pallas import tpu as pltpu
from jax.experimental.pallas import tpu_sc as plsc

VOCAB = 100000
EMBED = 64
OUT = 5
B = 16384
L = 50
DP = 8

NC = 2
NS = 16
NW = NC * NS
CHUNK = L
ROWS_PW = B // NW
CHUNKS_PW = ROWS_PW
NBUF = 8
GROUPS = CHUNKS_PW // NBUF


def _proj_body(t_ref, w_ref, z_ref):
    zz = jax.lax.dot_general(
        t_ref[...], w_ref[...],
        dimension_numbers=(((1,), (1,)), ((), ())),
        preferred_element_type=jnp.float32)
    z3 = zz.reshape(_ROWS_BLK // 16, 16, DP)
    packed = jnp.concatenate([z3[:, a, :] for a in range(16)], axis=1)
    z_ref[...] = packed.reshape(1, _ROWS_BLK // 16, 128)


_ROWS_BLK = 2000

_project = pl.pallas_call(
    _proj_body,
    grid=(VOCAB // _ROWS_BLK,),
    in_specs=[
        pl.BlockSpec((_ROWS_BLK, EMBED), lambda i: (i, 0)),
        pl.BlockSpec((DP, EMBED), lambda i: (0, 0)),
    ],
    out_specs=pl.BlockSpec((1, _ROWS_BLK // 16, 128), lambda i: (i, 0, 0)),
    out_shape=jax.ShapeDtypeStruct(
        (VOCAB // _ROWS_BLK, _ROWS_BLK // 16, 128), jnp.float32),
)


_mesh = plsc.VectorSubcoreMesh(core_axis_name="c", subcore_axis_name="s")


@functools.partial(
    pl.kernel,
    out_type=jax.ShapeDtypeStruct((B, 16), jnp.float32),
    mesh=_mesh,
    scratch_types=[
        pltpu.VMEM((ROWS_PW, L), jnp.int32),
        pltpu.VMEM((NBUF * CHUNK, DP), jnp.float32),
        pltpu.VMEM((ROWS_PW, 16), jnp.float32),
        pltpu.VMEM((16,), jnp.float32),
        pltpu.VMEM_SHARED((VOCAB, DP), jnp.float32),
        pltpu.SemaphoreType.DMA,
        [pltpu.SemaphoreType.DMA] * NBUF,
    ],
    compiler_params=pltpu.CompilerParams(use_tc_tiling_on_sc=False, needs_layout_passes=False),
)
def _sc_pool(idx_hbm, z_hbm, lb_hbm, out_hbm, idx_v, gbuf, out_v, lb_v,
             zsh, sem_i, gsems):
    sid = lax.axis_index("s")
    wid = sid * NC + lax.axis_index("c")
    rbase = wid * ROWS_PW

    zrows = VOCAB // NS
    pltpu.sync_copy(z_hbm.at[pl.ds(sid * zrows, zrows)],
                    zsh.at[pl.ds(sid * zrows, zrows)])
    pltpu.sync_copy(lb_hbm, lb_v)
    pltpu.sync_copy(idx_hbm.at[pl.ds(rbase, ROWS_PW)], idx_v)
    plsc.subcore_barrier()

    def _src(j):
        return zsh.at[idx_v.at[j]]

    def _dst(k):
        return gbuf.at[pl.ds(k * CHUNK, CHUNK)]

    for k in range(NBUF):
        pltpu.async_copy(_src(k), _dst(k), gsems[k])

    lane = lax.iota(jnp.int32, 16)
    half = lax.shift_right_logical(lane, 3)
    col = lane & 7
    fold = (lane + 8) & 15

    def _sum_rows(k, lo, hi):
        vals = [
            plsc.load_gather(gbuf, [k * CHUNK + 2 * i + half, col])
            for i in range(lo, hi)
        ]
        while len(vals) > 1:
            nxt = [vals[i] + vals[i + 1] for i in range(0, len(vals) - 1, 2)]
            if len(vals) % 2:
                nxt.append(vals[-1])
            vals = nxt
        s = vals[0]
        return s + jnp.take(s, fold, axis=0)

    def _group(g, carry):
        for k in range(NBUF):
            j = g * NBUF + k
            pltpu.make_async_copy(_src(k), _dst(k), gsems[k]).wait()
            out_v[j, :] = _sum_rows(k, 0, L // 2) + lb_v[...]

            @pl.when(j + NBUF < CHUNKS_PW)
            def _():
                pltpu.async_copy(_src(j + NBUF), _dst(k), gsems[k])
        return carry

    lax.fori_loop(0, GROUPS, _group, 0)

    pltpu.sync_copy(out_v, out_hbm.at[pl.ds(rbase, ROWS_PW)])


def kernel(inputs, table, W, b):
    W_pad = jnp.zeros((DP, EMBED), jnp.float32).at[:OUT].set(W)
    lb = jnp.zeros((16,), jnp.float32).at[:OUT].set(jnp.float32(L) * b)
    z = _project(table, W_pad).reshape(VOCAB, DP)
    pooled = _sc_pool(inputs, z, lb)
    return pooled[:, :OUT]


# speedup vs baseline: 1.2338x; 1.0565x over previous
import functools

import jax
import jax.numpy as jnp
from jax import lax
from jax.experimental import pallas as pl
from jax.experimental.pallas import tpu as pltpu
from jax.experimental.pallas import tpu_sc as plsc

VOCAB = 100000
EMBED = 64
OUT = 5
B = 16384
L = 50
DP = 8

NC = 2
NS = 16
NW = NC * NS
CHUNK = 56
ROWS_PW = B // NW
CHUNKS_PW = ROWS_PW
NBUF = 8
GROUPS = CHUNKS_PW // NBUF


def _proj_body(t_ref, w_ref, z_ref):
    zz = jax.lax.dot_general(
        t_ref[...], w_ref[...],
        dimension_numbers=(((1,), (1,)), ((), ())),
        preferred_element_type=jnp.float32)
    z3 = zz.reshape(_ROWS_BLK // 16, 16, DP)
    packed = jnp.concatenate([z3[:, a, :] for a in range(16)], axis=1)
    z_ref[...] = packed.reshape(1, _ROWS_BLK // 16, 128)


_ROWS_BLK = 10000

_project = pl.pallas_call(
    _proj_body,
    grid=(VOCAB // _ROWS_BLK,),
    in_specs=[
        pl.BlockSpec((_ROWS_BLK, EMBED), lambda i: (i, 0)),
        pl.BlockSpec((DP, EMBED), lambda i: (0, 0)),
    ],
    out_specs=pl.BlockSpec((1, _ROWS_BLK // 16, 128), lambda i: (i, 0, 0)),
    out_shape=jax.ShapeDtypeStruct(
        (VOCAB // _ROWS_BLK, _ROWS_BLK // 16, 128), jnp.float32),
)


_mesh = plsc.VectorSubcoreMesh(core_axis_name="c", subcore_axis_name="s")


@functools.partial(
    pl.kernel,
    out_type=jax.ShapeDtypeStruct((B, 16), jnp.float32),
    mesh=_mesh,
    scratch_types=[
        pltpu.VMEM((ROWS_PW, 128), jnp.int32),
        pltpu.VMEM((NBUF * CHUNK, DP), jnp.float32),
        pltpu.VMEM((ROWS_PW, 16), jnp.float32),
        pltpu.VMEM((16,), jnp.float32),
        pltpu.VMEM_SHARED((VOCAB, DP), jnp.float32),
        pltpu.SemaphoreType.DMA,
        [pltpu.SemaphoreType.DMA] * NBUF,
    ],
    compiler_params=pltpu.CompilerParams(use_tc_tiling_on_sc=False, needs_layout_passes=False),
)
def _sc_pool(idx_hbm, z_hbm, lb_hbm, out_hbm, idx_v, gbuf, out_v, lb_v,
             zsh, sem_i, gsems):
    sid = lax.axis_index("s")
    wid = sid * NC + lax.axis_index("c")
    rbase = wid * ROWS_PW

    zrows = VOCAB // NS
    pltpu.sync_copy(z_hbm.at[pl.ds(sid * zrows, zrows)],
                    zsh.at[pl.ds(sid * zrows, zrows)])
    pltpu.sync_copy(lb_hbm, lb_v)
    pltpu.sync_copy(idx_hbm.at[pl.ds(rbase, ROWS_PW)], idx_v)
    plsc.subcore_barrier()

    def _src(j):
        return zsh.at[idx_v.at[j, pl.ds(0, CHUNK)]]

    def _dst(k):
        return gbuf.at[pl.ds(k * CHUNK, CHUNK)]

    for k in range(NBUF):
        pltpu.async_copy(_src(k), _dst(k), gsems[k])

    lane = lax.iota(jnp.int32, 16)
    half = lax.shift_right_logical(lane, 3)
    col = lane & 7
    fold = (lane + 8) & 15

    def _sum_rows(k, lo, hi):
        vals = [
            plsc.load_gather(gbuf, [k * CHUNK + 2 * i + half, col])
            for i in range(lo, hi)
        ]
        while len(vals) > 1:
            nxt = [vals[i] + vals[i + 1] for i in range(0, len(vals) - 1, 2)]
            if len(vals) % 2:
                nxt.append(vals[-1])
            vals = nxt
        s = vals[0]
        return s + jnp.take(s, fold, axis=0)

    def _group(g, carry):
        for k in range(NBUF):
            j = g * NBUF + k
            pltpu.make_async_copy(_src(k), _dst(k), gsems[k]).wait()
            out_v[j, :] = _sum_rows(k, 0, L // 2) + lb_v[...]

            @pl.when(j + NBUF < CHUNKS_PW)
            def _():
                pltpu.async_copy(_src(j + NBUF), _dst(k), gsems[k])
        return carry

    lax.fori_loop(0, GROUPS, _group, 0)

    pltpu.sync_copy(out_v, out_hbm.at[pl.ds(rbase, ROWS_PW)])


def kernel(inputs, table, W, b):
    W_pad = jnp.zeros((DP, EMBED), jnp.float32).at[:OUT].set(W)
    lb = jnp.zeros((16,), jnp.float32).at[:OUT].set(jnp.float32(L) * b)
    z = _project(table, W_pad).reshape(VOCAB, DP)
    idx_pad = jnp.pad(inputs, ((0, 0), (0, 128 - L)))
    pooled = _sc_pool(idx_pad, z, lb)
    return pooled[:, :OUT]
